# Initial kernel scaffold; baseline (speedup 1.0000x reference)
import jax
import jax.numpy as jnp
from jax.experimental import pallas as pl


def kernel(actions, rewards):
    raise NotImplementedError("write your pallas kernel here")


# speedup vs baseline: 2.0218x; 2.0218x over previous
import jax
import jax.numpy as jnp
from jax.experimental import pallas as pl
from jax.experimental.pallas import tpu as pltpu

NUM_CANDIDATES = 1024
NUM_ENVS = 128
NUM_HORIZON = 16
ACTION_DIM = 8
NUM_TOP = 128
HD = NUM_HORIZON * ACTION_DIM
ACTION_LOW = -1.0
ACTION_HIGH = 1.0

_CAND_BLK = 128
_N_BLKS = NUM_CANDIDATES // _CAND_BLK

_EPS_CACHE = []


def _eps_const():
    if not _EPS_CACHE:
        e = jax.random.normal(
            jax.random.key(1),
            (NUM_CANDIDATES, NUM_ENVS, NUM_HORIZON, ACTION_DIM),
            dtype=jnp.float32,
        ).reshape(NUM_CANDIDATES, NUM_ENVS, HD)
        _EPS_CACHE.append(e)
    return _EPS_CACHE[0]


def _select_kernel(r_ref, mask_ref):
    r = r_ref[...]
    lane = jax.lax.broadcasted_iota(jnp.int32, (NUM_ENVS * NUM_HORIZON, NUM_ENVS), 0)
    env = jax.lax.broadcasted_iota(jnp.int32, (NUM_ENVS * NUM_HORIZON, NUM_ENVS), 1)
    sel_mat = jnp.where((lane // NUM_HORIZON) == env, jnp.float32(1), jnp.float32(0))
    s = jax.lax.dot(r, sel_mat, precision=jax.lax.Precision.HIGHEST)

    u = jax.lax.bitcast_convert_type(s, jnp.int32)
    key = u ^ ((u >> 31) & jnp.int32(0x7FFFFFFF))

    nonneg_i = jnp.where(key >= 0, jnp.int32(1), jnp.int32(0))
    cnt_ge0 = jnp.sum(nonneg_i, axis=0, keepdims=True)
    pos_branch = cnt_ge0 >= NUM_TOP
    domain_i = jnp.where(pos_branch, nonneg_i, 1 - nonneg_i)
    kth = jnp.where(pos_branch, NUM_TOP, NUM_TOP - cnt_ge0)
    v = key & jnp.int32(0x7FFFFFFF)

    def body(i, t):
        bit = jnp.int32(1) << (jnp.int32(30) - i)
        t2 = t | bit
        c = jnp.sum(domain_i * jnp.where(v >= t2, jnp.int32(1), jnp.int32(0)),
                    axis=0, keepdims=True)
        return jnp.where(c >= kth, t2, t)

    t0 = jnp.zeros((1, NUM_ENVS), jnp.int32)
    kv = jax.lax.fori_loop(0, 31, body, t0)
    k_full = jnp.where(pos_branch, kv, kv | jnp.int32(-0x80000000))

    gt = key > k_full
    tie = key == k_full
    n_gt = jnp.sum(jnp.where(gt, jnp.int32(1), jnp.int32(0)), axis=0, keepdims=True)
    need = (NUM_TOP - n_gt).astype(jnp.float32)

    row = jax.lax.broadcasted_iota(jnp.int32, (NUM_CANDIDATES, NUM_CANDIDATES), 0)
    col = jax.lax.broadcasted_iota(jnp.int32, (NUM_CANDIDATES, NUM_CANDIDATES), 1)
    tri = jnp.where(col <= row, jnp.float32(1), jnp.float32(0))
    tie_f = jnp.where(tie, jnp.float32(1), jnp.float32(0))
    tie_rank = jax.lax.dot(tri, tie_f, precision=jax.lax.Precision.HIGHEST)

    sel = jnp.where(gt, 1.0, 0.0) + jnp.where(tie & (tie_rank <= need), 1.0, 0.0)
    mask_ref[...] = sel


def _moments_kernel(a_ref, m_ref, mu_ref, std_ref, acc_ref, acc2_ref):
    i = pl.program_id(0)

    @pl.when(i == 0)
    def _init():
        acc_ref[...] = jnp.zeros_like(acc_ref)
        acc2_ref[...] = jnp.zeros_like(acc2_ref)

    a = a_ref[...]
    w = m_ref[...][:, :, None]
    aw = a * w
    acc_ref[...] += jnp.sum(aw, axis=0)
    acc2_ref[...] += jnp.sum(aw * a, axis=0)

    @pl.when(i == _N_BLKS - 1)
    def _fin():
        inv = jnp.float32(1.0 / NUM_TOP)
        mu = acc_ref[...] * inv
        var = acc2_ref[...] * inv - mu * mu
        std = jnp.sqrt(jnp.maximum(var, 0.0))
        mu_ref[...] = mu
        std_ref[...] = jnp.maximum(std, 1e-6)


def _sample_kernel(mu_ref, std_ref, e_ref, o_ref):
    mu = mu_ref[...][None]
    std = std_ref[...][None]
    o = mu + std * e_ref[...]
    o_ref[...] = jnp.clip(o, ACTION_LOW, ACTION_HIGH)


def kernel(actions, rewards):
    a3 = actions.reshape(NUM_CANDIDATES, NUM_ENVS, HD)
    r2 = rewards.reshape(NUM_CANDIDATES, NUM_ENVS * NUM_HORIZON)
    eps = _eps_const()

    mask = pl.pallas_call(
        _select_kernel,
        out_shape=jax.ShapeDtypeStruct((NUM_CANDIDATES, NUM_ENVS), jnp.float32),
    )(r2)

    mu, std = pl.pallas_call(
        _moments_kernel,
        grid=(_N_BLKS,),
        in_specs=[
            pl.BlockSpec((_CAND_BLK, NUM_ENVS, HD), lambda i: (i, 0, 0)),
            pl.BlockSpec((_CAND_BLK, NUM_ENVS), lambda i: (i, 0)),
        ],
        out_specs=[
            pl.BlockSpec((NUM_ENVS, HD), lambda i: (0, 0)),
            pl.BlockSpec((NUM_ENVS, HD), lambda i: (0, 0)),
        ],
        out_shape=[
            jax.ShapeDtypeStruct((NUM_ENVS, HD), jnp.float32),
            jax.ShapeDtypeStruct((NUM_ENVS, HD), jnp.float32),
        ],
        scratch_shapes=[
            pltpu.VMEM((NUM_ENVS, HD), jnp.float32),
            pltpu.VMEM((NUM_ENVS, HD), jnp.float32),
        ],
    )(a3, mask)

    out = pl.pallas_call(
        _sample_kernel,
        grid=(_N_BLKS,),
        in_specs=[
            pl.BlockSpec((NUM_ENVS, HD), lambda i: (0, 0)),
            pl.BlockSpec((NUM_ENVS, HD), lambda i: (0, 0)),
            pl.BlockSpec((_CAND_BLK, NUM_ENVS, HD), lambda i: (i, 0, 0)),
        ],
        out_specs=pl.BlockSpec((_CAND_BLK, NUM_ENVS, HD), lambda i: (i, 0, 0)),
        out_shape=jax.ShapeDtypeStruct((NUM_CANDIDATES, NUM_ENVS, HD), jnp.float32),
    )(mu, std, eps)

    return out.reshape(NUM_CANDIDATES, NUM_ENVS, NUM_HORIZON, ACTION_DIM)


# speedup vs baseline: 2.0218x; 1.0000x over previous
import jax
import jax.numpy as jnp
from jax.experimental import pallas as pl
from jax.experimental.pallas import tpu as pltpu

NUM_CANDIDATES = 1024
NUM_ENVS = 128
NUM_HORIZON = 16
ACTION_DIM = 8
NUM_TOP = 128
HD = NUM_HORIZON * ACTION_DIM
ACTION_LOW = -1.0
ACTION_HIGH = 1.0

_CAND_BLK = 128
_N_BLKS = NUM_CANDIDATES // _CAND_BLK

def _select_kernel(r_ref, mask_ref):
    r = r_ref[...]
    lane = jax.lax.broadcasted_iota(jnp.int32, (NUM_ENVS * NUM_HORIZON, NUM_ENVS), 0)
    env = jax.lax.broadcasted_iota(jnp.int32, (NUM_ENVS * NUM_HORIZON, NUM_ENVS), 1)
    sel_mat = jnp.where((lane // NUM_HORIZON) == env, jnp.float32(1), jnp.float32(0))
    s = jax.lax.dot(r, sel_mat, precision=jax.lax.Precision.HIGHEST)

    u = jax.lax.bitcast_convert_type(s, jnp.int32)
    key = u ^ ((u >> 31) & jnp.int32(0x7FFFFFFF))

    nonneg_i = jnp.where(key >= 0, jnp.int32(1), jnp.int32(0))
    cnt_ge0 = jnp.sum(nonneg_i, axis=0, keepdims=True)
    pos_branch = cnt_ge0 >= NUM_TOP
    domain_i = jnp.where(pos_branch, nonneg_i, 1 - nonneg_i)
    kth = jnp.where(pos_branch, NUM_TOP, NUM_TOP - cnt_ge0)
    v = key & jnp.int32(0x7FFFFFFF)

    def body(i, t):
        bit = jnp.int32(1) << (jnp.int32(30) - i)
        t2 = t | bit
        c = jnp.sum(domain_i * jnp.where(v >= t2, jnp.int32(1), jnp.int32(0)),
                    axis=0, keepdims=True)
        return jnp.where(c >= kth, t2, t)

    t0 = jnp.zeros((1, NUM_ENVS), jnp.int32)
    kv = jax.lax.fori_loop(0, 31, body, t0)
    k_full = jnp.where(pos_branch, kv, kv | jnp.int32(-0x80000000))

    gt = key > k_full
    tie = key == k_full
    n_gt = jnp.sum(jnp.where(gt, jnp.int32(1), jnp.int32(0)), axis=0, keepdims=True)
    need = (NUM_TOP - n_gt).astype(jnp.float32)

    row = jax.lax.broadcasted_iota(jnp.int32, (NUM_CANDIDATES, NUM_CANDIDATES), 0)
    col = jax.lax.broadcasted_iota(jnp.int32, (NUM_CANDIDATES, NUM_CANDIDATES), 1)
    tri = jnp.where(col <= row, jnp.float32(1), jnp.float32(0))
    tie_f = jnp.where(tie, jnp.float32(1), jnp.float32(0))
    tie_rank = jax.lax.dot(tri, tie_f, precision=jax.lax.Precision.HIGHEST)

    sel = jnp.where(gt, 1.0, 0.0) + jnp.where(tie & (tie_rank <= need), 1.0, 0.0)
    mask_ref[...] = sel


def _moments_kernel(a_ref, m_ref, mu_ref, std_ref, acc_ref, acc2_ref):
    i = pl.program_id(0)

    @pl.when(i == 0)
    def _init():
        acc_ref[...] = jnp.zeros_like(acc_ref)
        acc2_ref[...] = jnp.zeros_like(acc2_ref)

    a = a_ref[...]
    w = m_ref[...][:, :, None]
    aw = a * w
    acc_ref[...] += jnp.sum(aw, axis=0)
    acc2_ref[...] += jnp.sum(aw * a, axis=0)

    @pl.when(i == _N_BLKS - 1)
    def _fin():
        inv = jnp.float32(1.0 / NUM_TOP)
        mu = acc_ref[...] * inv
        var = acc2_ref[...] * inv - mu * mu
        std = jnp.sqrt(jnp.maximum(var, 0.0))
        mu_ref[...] = mu
        std_ref[...] = jnp.maximum(std, 1e-6)


def _sample_kernel(mu_ref, std_ref, e_ref, o_ref):
    mu = mu_ref[...][None]
    std = std_ref[...][None]
    o = mu + std * e_ref[...]
    o_ref[...] = jnp.clip(o, ACTION_LOW, ACTION_HIGH)


def kernel(actions, rewards):
    a3 = actions.reshape(NUM_CANDIDATES, NUM_ENVS, HD)
    r2 = rewards.reshape(NUM_CANDIDATES, NUM_ENVS * NUM_HORIZON)
    eps = jax.random.normal(
        jax.random.key(1),
        (NUM_CANDIDATES, NUM_ENVS, NUM_HORIZON, ACTION_DIM),
        dtype=jnp.float32,
    ).reshape(NUM_CANDIDATES, NUM_ENVS, HD)

    mask = pl.pallas_call(
        _select_kernel,
        out_shape=jax.ShapeDtypeStruct((NUM_CANDIDATES, NUM_ENVS), jnp.float32),
    )(r2)

    mu, std = pl.pallas_call(
        _moments_kernel,
        grid=(_N_BLKS,),
        in_specs=[
            pl.BlockSpec((_CAND_BLK, NUM_ENVS, HD), lambda i: (i, 0, 0)),
            pl.BlockSpec((_CAND_BLK, NUM_ENVS), lambda i: (i, 0)),
        ],
        out_specs=[
            pl.BlockSpec((NUM_ENVS, HD), lambda i: (0, 0)),
            pl.BlockSpec((NUM_ENVS, HD), lambda i: (0, 0)),
        ],
        out_shape=[
            jax.ShapeDtypeStruct((NUM_ENVS, HD), jnp.float32),
            jax.ShapeDtypeStruct((NUM_ENVS, HD), jnp.float32),
        ],
        scratch_shapes=[
            pltpu.VMEM((NUM_ENVS, HD), jnp.float32),
            pltpu.VMEM((NUM_ENVS, HD), jnp.float32),
        ],
    )(a3, mask)

    out = pl.pallas_call(
        _sample_kernel,
        grid=(_N_BLKS,),
        in_specs=[
            pl.BlockSpec((NUM_ENVS, HD), lambda i: (0, 0)),
            pl.BlockSpec((NUM_ENVS, HD), lambda i: (0, 0)),
            pl.BlockSpec((_CAND_BLK, NUM_ENVS, HD), lambda i: (i, 0, 0)),
        ],
        out_specs=pl.BlockSpec((_CAND_BLK, NUM_ENVS, HD), lambda i: (i, 0, 0)),
        out_shape=jax.ShapeDtypeStruct((NUM_CANDIDATES, NUM_ENVS, HD), jnp.float32),
    )(mu, std, eps)

    return out.reshape(NUM_CANDIDATES, NUM_ENVS, NUM_HORIZON, ACTION_DIM)


# speedup vs baseline: 2.4946x; 1.2339x over previous
import jax
import jax.numpy as jnp
from jax.experimental import pallas as pl
from jax.experimental.pallas import tpu as pltpu

NUM_CANDIDATES = 1024
NUM_ENVS = 128
NUM_HORIZON = 16
ACTION_DIM = 8
NUM_TOP = 128
HD = NUM_HORIZON * ACTION_DIM
ACTION_LOW = -1.0
ACTION_HIGH = 1.0

_CAND_BLK = 128
_N_BLKS = NUM_CANDIDATES // _CAND_BLK


def _select_kernel(r_ref, mask_ref):
    s = r_ref[:, 0, :]
    for h in range(1, NUM_HORIZON):
        s = s + r_ref[:, h, :]

    u = jax.lax.bitcast_convert_type(s, jnp.int32)
    key = u ^ ((u >> 31) & jnp.int32(0x7FFFFFFF))

    nonneg_i = jnp.where(key >= 0, jnp.int32(1), jnp.int32(0))
    cnt_ge0 = jnp.sum(nonneg_i, axis=0, keepdims=True)
    pos_branch = cnt_ge0 >= NUM_TOP
    domain_i = jnp.where(pos_branch, nonneg_i, 1 - nonneg_i)
    kth = jnp.where(pos_branch, NUM_TOP, NUM_TOP - cnt_ge0)
    v = key & jnp.int32(0x7FFFFFFF)

    def body(i, t):
        bit = jnp.int32(1) << (jnp.int32(30) - i)
        t2 = t | bit
        c = jnp.sum(domain_i * jnp.where(v >= t2, jnp.int32(1), jnp.int32(0)),
                    axis=0, keepdims=True)
        return jnp.where(c >= kth, t2, t)

    t0 = jnp.zeros((1, NUM_ENVS), jnp.int32)
    kv = jax.lax.fori_loop(0, 31, body, t0)
    k_full = jnp.where(pos_branch, kv, kv | jnp.int32(-0x80000000))

    gt = key > k_full
    tie = key == k_full
    n_gt = jnp.sum(jnp.where(gt, jnp.int32(1), jnp.int32(0)), axis=0, keepdims=True)
    need = (NUM_TOP - n_gt).astype(jnp.float32)

    row = jax.lax.broadcasted_iota(jnp.int32, (NUM_CANDIDATES, NUM_CANDIDATES), 0)
    col = jax.lax.broadcasted_iota(jnp.int32, (NUM_CANDIDATES, NUM_CANDIDATES), 1)
    tri = jnp.where(col <= row, jnp.float32(1), jnp.float32(0))
    tie_f = jnp.where(tie, jnp.float32(1), jnp.float32(0))
    tie_rank = jax.lax.dot(tri, tie_f, precision=jax.lax.Precision.HIGHEST)

    sel = jnp.where(gt, 1.0, 0.0) + jnp.where(tie & (tie_rank <= need), 1.0, 0.0)
    mask_ref[...] = sel


def _moments_kernel(a_ref, m_ref, mu_ref, std_ref, acc_ref, acc2_ref):
    i = pl.program_id(0)

    @pl.when(i == 0)
    def _init():
        acc_ref[...] = jnp.zeros_like(acc_ref)
        acc2_ref[...] = jnp.zeros_like(acc2_ref)

    a = a_ref[...]
    w = m_ref[...][:, None, :]
    aw = a * w
    acc_ref[...] += jnp.sum(aw, axis=0)
    acc2_ref[...] += jnp.sum(aw * a, axis=0)

    @pl.when(i == _N_BLKS - 1)
    def _fin():
        inv = jnp.float32(1.0 / NUM_TOP)
        mu = acc_ref[...] * inv
        var = acc2_ref[...] * inv - mu * mu
        std = jnp.sqrt(jnp.maximum(var, 0.0))
        mu_ref[...] = mu
        std_ref[...] = jnp.maximum(std, 1e-6)


def _sample_kernel(mu_ref, std_ref, e_ref, o_ref):
    mu = mu_ref[...][None]
    std = std_ref[...][None]
    o = mu + std * e_ref[...]
    o_ref[...] = jnp.clip(o, ACTION_LOW, ACTION_HIGH)


def kernel(actions, rewards):
    a3 = actions.transpose(0, 2, 3, 1).reshape(NUM_CANDIDATES, HD, NUM_ENVS)
    r3 = rewards.transpose(0, 2, 3, 1).reshape(NUM_CANDIDATES, NUM_HORIZON, NUM_ENVS)
    eps = jax.random.normal(
        jax.random.key(1),
        (NUM_CANDIDATES, NUM_ENVS, NUM_HORIZON, ACTION_DIM),
        dtype=jnp.float32,
    ).transpose(0, 2, 3, 1).reshape(NUM_CANDIDATES, HD, NUM_ENVS)

    mask = pl.pallas_call(
        _select_kernel,
        out_shape=jax.ShapeDtypeStruct((NUM_CANDIDATES, NUM_ENVS), jnp.float32),
    )(r3)

    mu, std = pl.pallas_call(
        _moments_kernel,
        grid=(_N_BLKS,),
        in_specs=[
            pl.BlockSpec((_CAND_BLK, HD, NUM_ENVS), lambda i: (i, 0, 0)),
            pl.BlockSpec((_CAND_BLK, NUM_ENVS), lambda i: (i, 0)),
        ],
        out_specs=[
            pl.BlockSpec((HD, NUM_ENVS), lambda i: (0, 0)),
            pl.BlockSpec((HD, NUM_ENVS), lambda i: (0, 0)),
        ],
        out_shape=[
            jax.ShapeDtypeStruct((HD, NUM_ENVS), jnp.float32),
            jax.ShapeDtypeStruct((HD, NUM_ENVS), jnp.float32),
        ],
        scratch_shapes=[
            pltpu.VMEM((HD, NUM_ENVS), jnp.float32),
            pltpu.VMEM((HD, NUM_ENVS), jnp.float32),
        ],
    )(a3, mask)

    out = pl.pallas_call(
        _sample_kernel,
        grid=(_N_BLKS,),
        in_specs=[
            pl.BlockSpec((HD, NUM_ENVS), lambda i: (0, 0)),
            pl.BlockSpec((HD, NUM_ENVS), lambda i: (0, 0)),
            pl.BlockSpec((_CAND_BLK, HD, NUM_ENVS), lambda i: (i, 0, 0)),
        ],
        out_specs=pl.BlockSpec((_CAND_BLK, HD, NUM_ENVS), lambda i: (i, 0, 0)),
        out_shape=jax.ShapeDtypeStruct((NUM_CANDIDATES, HD, NUM_ENVS), jnp.float32),
    )(mu, std, eps)

    return (out.reshape(NUM_CANDIDATES, NUM_HORIZON, ACTION_DIM, NUM_ENVS)
               .transpose(0, 3, 1, 2))
